# baseline (speedup 1.0000x reference)
import functools

import jax
import jax.numpy as jnp
from jax import lax
from jax.experimental import pallas as pl
from jax.experimental.pallas import tpu as pltpu
from jax.experimental.pallas import tpu_sc as plsc

_CHUNK = 128


@jax.jit
def kernel(speaker_ids, embed_weight):
    B = speaker_ids.shape[0]
    V, D = embed_weight.shape

    info = plsc.get_sparse_core_info()
    NC, NS = info.num_cores, info.num_subcores
    NW = NC * NS

    b_per_w = B // NW
    n_ch = b_per_w // _CHUNK

    ids = speaker_ids.astype(jnp.int32)
    idx_half = (ids >> 1).reshape(NW, n_ch, _CHUNK)
    table2 = embed_weight.reshape(V // 2, 2 * D)

    mesh = plsc.VectorSubcoreMesh(core_axis_name="c", subcore_axis_name="s")

    @functools.partial(
        pl.kernel,
        out_type=jax.ShapeDtypeStruct((B, 2 * D), jnp.float32),
        mesh=mesh,
        compiler_params=pltpu.CompilerParams(use_tc_tiling_on_sc=True),
        scratch_types=[
            pltpu.VMEM((n_ch, _CHUNK), jnp.int32),
            pltpu.VMEM((b_per_w, 2 * D), jnp.float32),
            pltpu.SemaphoreType.DMA,
        ],
    )
    def gather_kernel(idx_hbm, table_hbm, out_hbm, idx_v, rows_v, sem):
        wid = lax.axis_index("s") * NC + lax.axis_index("c")
        base = pl.multiple_of(wid * b_per_w, 128)
        pltpu.sync_copy(idx_hbm.at[wid], idx_v)
        copies = []
        for j in range(n_ch):
            copies.append(
                pltpu.async_copy(
                    table_hbm.at[idx_v.at[j]],
                    rows_v.at[pl.ds(j * _CHUNK, _CHUNK)],
                    sem,
                )
            )
        for c in copies:
            c.wait()
        pltpu.sync_copy(rows_v, out_hbm.at[pl.ds(base, b_per_w)])

    wide = gather_kernel(idx_half, table2)
    return jnp.where((ids & 1)[:, None] == 1, wide[:, D:], wide[:, :D])


# speedup vs baseline: 1.8911x; 1.8911x over previous
import functools

import jax
import jax.numpy as jnp
from jax import lax
from jax.experimental import pallas as pl
from jax.experimental.pallas import tpu as pltpu
from jax.experimental.pallas import tpu_sc as plsc


@jax.jit
def kernel(speaker_ids, embed_weight):
    B = speaker_ids.shape[0]
    V, D = embed_weight.shape

    info = plsc.get_sparse_core_info()
    NC, NS = info.num_cores, info.num_subcores
    NW = NC * NS

    b_per_w = B // NW
    n_grp = b_per_w // 16

    ids = speaker_ids.astype(jnp.int32).reshape(NW, 1, b_per_w)
    table_t = embed_weight.T

    mesh = plsc.VectorSubcoreMesh(core_axis_name="c", subcore_axis_name="s")

    @functools.partial(
        pl.kernel,
        out_type=jax.ShapeDtypeStruct((NW, D, b_per_w), jnp.float32),
        mesh=mesh,
        compiler_params=pltpu.CompilerParams(
            use_tc_tiling_on_sc=True,
            disable_bounds_checks=True,
            needs_layout_passes=False,
        ),
        scratch_types=[
            pltpu.VMEM((1, b_per_w), jnp.int32),
            pltpu.VMEM((2, D, 128), jnp.float32),
            pltpu.VMEM((D, b_per_w), jnp.float32),
            pltpu.SemaphoreType.DMA((2,)),
        ],
    )
    def gather_kernel(idx_hbm, table_hbm, out_hbm, idx_v, blk, buf, sem):
        wid = lax.axis_index("s") * NC + lax.axis_index("c")
        pltpu.sync_copy(idx_hbm.at[wid], idx_v)

        iota = lax.iota(jnp.int32, 16)

        def copy_for(c, par):
            cb = pl.multiple_of((c >> 7) << 7, 128)
            return pltpu.make_async_copy(
                table_hbm.at[:, pl.ds(cb, 128)],
                blk.at[par],
                sem.at[par],
            )

        first = idx_v[0, pl.ds(0, 16)]
        copy_for(first[0], 0).start()

        def body(g, cur):
            nxt = idx_v[0, pl.ds(((g + 1) % n_grp) * 16, 16)]
            for t in range(16):
                m = g * 16 + t
                c = cur[t]
                c_next = cur[t + 1] if t < 15 else nxt[0]

                @pl.when(m + 1 < b_per_w)
                def _():
                    copy_for(c_next, (t + 1) & 1).start()

                copy_for(c, t & 1).wait()
                lane = jnp.full((16,), c & 127, jnp.int32)
                col = jnp.full((16,), m, jnp.int32)
                src = blk.at[t & 1]
                for k in range(D // 16):
                    rows = iota + (16 * k)
                    v = plsc.load_gather(src, [rows, lane])
                    plsc.store_scatter(buf, [rows, col], v)
            return nxt

        lax.fori_loop(0, n_grp, body, first)
        pltpu.sync_copy(buf, out_hbm.at[wid])

    blocks = gather_kernel(ids, table_t)
    return blocks.transpose(0, 2, 1).reshape(B, D)


# speedup vs baseline: 2.6302x; 1.3909x over previous
import functools

import jax
import jax.numpy as jnp
from jax import lax
from jax.experimental import pallas as pl
from jax.experimental.pallas import tpu as pltpu
from jax.experimental.pallas import tpu_sc as plsc

_NBUF = 4


@jax.jit
def kernel(speaker_ids, embed_weight):
    B = speaker_ids.shape[0]
    V, D = embed_weight.shape

    info = plsc.get_sparse_core_info()
    NC, NS = info.num_cores, info.num_subcores
    NW = NC * NS

    b_per_w = B // NW
    n_grp = b_per_w // 16

    ids = speaker_ids.astype(jnp.int32).reshape(NW, 1, b_per_w)
    table_t = embed_weight.T

    mesh = plsc.VectorSubcoreMesh(core_axis_name="c", subcore_axis_name="s")

    @functools.partial(
        pl.kernel,
        out_type=jax.ShapeDtypeStruct((D, B), jnp.float32),
        mesh=mesh,
        compiler_params=pltpu.CompilerParams(
            use_tc_tiling_on_sc=True,
            disable_bounds_checks=True,
            needs_layout_passes=False,
        ),
        scratch_types=[
            pltpu.VMEM((1, b_per_w), jnp.int32),
            pltpu.VMEM((_NBUF, D, 128), jnp.float32),
            pltpu.VMEM((D, b_per_w), jnp.float32),
            pltpu.SemaphoreType.DMA((_NBUF,)),
        ],
    )
    def gather_kernel(idx_hbm, table_hbm, out_hbm, idx_v, blk, buf, sem):
        wid = lax.axis_index("s") * NC + lax.axis_index("c")
        base = pl.multiple_of(wid * b_per_w, 128)
        pltpu.sync_copy(idx_hbm.at[wid], idx_v)

        iota = lax.iota(jnp.int32, 16)

        def copy_for(c, par):
            cb = pl.multiple_of((c >> 7) << 7, 128)
            return pltpu.make_async_copy(
                table_hbm.at[:, pl.ds(cb, 128)],
                blk.at[par],
                sem.at[par],
            )

        first = idx_v[0, pl.ds(0, 16)]
        for t in range(_NBUF - 1):
            copy_for(first[t], t).start()

        def body(g, cur):
            nxt = idx_v[0, pl.ds(((g + 1) % n_grp) * 16, 16)]
            for t in range(16):
                m = g * 16 + t
                ahead = t + _NBUF - 1
                c_next = cur[ahead] if ahead < 16 else nxt[ahead - 16]

                @pl.when(m + _NBUF - 1 < b_per_w)
                def _():
                    copy_for(c_next, ahead & (_NBUF - 1)).start()

                c = cur[t]
                copy_for(c, t & (_NBUF - 1)).wait()
                lane = jnp.full((16,), c & 127, jnp.int32)
                col = jnp.full((16,), m, jnp.int32)
                src = blk.at[t & (_NBUF - 1)]
                for k in range(D // 16):
                    rows = iota + (16 * k)
                    v = plsc.load_gather(src, [rows, lane])
                    plsc.store_scatter(buf, [rows, col], v)
            return nxt

        lax.fori_loop(0, n_grp, body, first)
        pltpu.sync_copy(buf, out_hbm.at[:, pl.ds(base, b_per_w)])

    out_t = gather_kernel(ids, table_t)
    return out_t.T


# speedup vs baseline: 3.0766x; 1.1697x over previous
import functools

import jax
import jax.numpy as jnp
from jax import lax
from jax.experimental import pallas as pl
from jax.experimental.pallas import tpu as pltpu
from jax.experimental.pallas import tpu_sc as plsc

_NBUF = 8


@jax.jit
def kernel(speaker_ids, embed_weight):
    B = speaker_ids.shape[0]
    V, D = embed_weight.shape

    info = plsc.get_sparse_core_info()
    NC, NS = info.num_cores, info.num_subcores
    NW = NC * NS

    b_per_w = B // NW
    n_grp = b_per_w // 16

    ids = speaker_ids.astype(jnp.int32).reshape(NW, 1, b_per_w)
    table_t = embed_weight.T

    mesh = plsc.VectorSubcoreMesh(core_axis_name="c", subcore_axis_name="s")

    @functools.partial(
        pl.kernel,
        out_type=jax.ShapeDtypeStruct((D, B), jnp.float32),
        mesh=mesh,
        compiler_params=pltpu.CompilerParams(
            use_tc_tiling_on_sc=True,
            disable_bounds_checks=True,
            needs_layout_passes=False,
        ),
        scratch_types=[
            pltpu.VMEM((1, b_per_w), jnp.int32),
            pltpu.VMEM((_NBUF, D, 128), jnp.float32),
            pltpu.VMEM((D, b_per_w), jnp.float32),
            pltpu.SemaphoreType.DMA((_NBUF,)),
        ],
    )
    def gather_kernel(idx_hbm, table_hbm, out_hbm, idx_v, blk, buf, sem):
        wid = lax.axis_index("s") * NC + lax.axis_index("c")
        base = pl.multiple_of(wid * b_per_w, 128)
        pltpu.sync_copy(idx_hbm.at[wid], idx_v)

        iota = lax.iota(jnp.int32, 16)

        def copy_for(c, par):
            cb = pl.multiple_of((c >> 7) << 7, 128)
            return pltpu.make_async_copy(
                table_hbm.at[:, pl.ds(cb, 128)],
                blk.at[par],
                sem.at[par],
            )

        first = idx_v[0, pl.ds(0, 16)]
        for t in range(_NBUF - 1):
            copy_for(first[t], t).start()

        def body(g, cur):
            nxt = idx_v[0, pl.ds(((g + 1) % n_grp) * 16, 16)]
            for t in range(16):
                m = g * 16 + t
                ahead = t + _NBUF - 1
                c_next = cur[ahead] if ahead < 16 else nxt[ahead - 16]

                @pl.when(m + _NBUF - 1 < b_per_w)
                def _():
                    copy_for(c_next, ahead & (_NBUF - 1)).start()

                c = cur[t]
                copy_for(c, t & (_NBUF - 1)).wait()
                lane = jnp.full((16,), c & 127, jnp.int32)
                col = jnp.full((16,), m, jnp.int32)
                src = blk.at[t & (_NBUF - 1)]
                for k in range(D // 16):
                    rows = iota + (16 * k)
                    v = plsc.load_gather(src, [rows, lane])
                    plsc.store_scatter(buf, [rows, col], v)
            return nxt

        lax.fori_loop(0, n_grp, body, first)
        pltpu.sync_copy(buf, out_hbm.at[:, pl.ds(base, b_per_w)])

    out_t = gather_kernel(ids, table_t)
    return out_t.T
